# baseline (speedup 1.0000x reference)
import functools

import jax
import jax.numpy as jnp
from jax.experimental import pallas as pl
from jax.experimental.pallas import tpu as pltpu

_BN_EPS = 1e-5
_PAD = 128
_CHUNKS = 4
_BB = 4


def _fused_block_kernel(x_ref, w_ref, b1_ref, pw_ref, b2_ref, o_ref,
                        xpad_ref, *, hw, w_img, kh, kw):
    c = x_ref.shape[1]
    ph, pw_pad = kh // 2, kw // 2
    margin = w_img * ph
    t = hw // _CHUNKS
    wk = t + 2 * margin

    b1 = b1_ref[...]
    wt = w_ref[...]
    pwb = pw_ref[...]
    b2 = b2_ref[...]

    col = jax.lax.broadcasted_iota(jnp.int32, (c, wk), 1) % w_img

    for bb in range(_BB):
        xpad_ref[:, :_PAD] = jnp.zeros((c, _PAD), jnp.bfloat16)
        xpad_ref[:, _PAD + hw:] = jnp.zeros((c, _PAD), jnp.bfloat16)
        xpad_ref[:, _PAD:_PAD + hw] = x_ref[bb].astype(jnp.bfloat16)

        for k in range(_CHUNKS):
            base = _PAD + k * t - margin

            us = []
            for j in range(kw):
                dj = j - pw_pad
                u = xpad_ref[:, base + dj:base + dj + wk]
                if dj < 0:
                    u = jnp.where(col >= -dj, u, jnp.bfloat16(0))
                elif dj > 0:
                    u = jnp.where(col < w_img - dj, u, jnp.bfloat16(0))
                us.append(u)

            acc = None
            for i in range(kh):
                v = None
                for j in range(kw):
                    term = us[j] * wt[:, kw * i + j:kw * i + j + 1]
                    v = term if v is None else v + term
                sh = w_img * i
                part = v[:, sh:sh + t]
                acc = part if acc is None else acc + part

            mid = jnp.maximum(acc + b1, jnp.bfloat16(0))
            y = jnp.dot(pwb, mid, preferred_element_type=jnp.float32)
            o_ref[bb, :, k * t:k * t + t] = (
                jnp.maximum(y + b2, 0.0).astype(o_ref.dtype))


def kernel(x, dw_w, pw_w, bn1_gamma, bn1_beta, bn1_mean, bn1_var,
           bn2_gamma, bn2_beta, bn2_mean, bn2_var):
    n, c_in, h, w = x.shape
    kh, kw = int(dw_w.shape[2]), int(dw_w.shape[3])
    c_out = pw_w.shape[0]
    hw = h * w

    s1 = bn1_gamma / jnp.sqrt(bn1_var + _BN_EPS)
    b1 = bn1_beta - bn1_mean * s1
    s2 = bn2_gamma / jnp.sqrt(bn2_var + _BN_EPS)
    b2 = bn2_beta - bn2_mean * s2
    w_taps = (dw_w.reshape(c_in, kh * kw) * s1[:, None]).astype(jnp.bfloat16)
    pw_folded = (pw_w.reshape(c_out, c_in) * s2[:, None]).astype(jnp.bfloat16)

    x_flat = x.reshape(n, c_in, hw)

    body = functools.partial(_fused_block_kernel, hw=hw, w_img=w, kh=kh, kw=kw)
    out_flat = pl.pallas_call(
        body,
        out_shape=jax.ShapeDtypeStruct((n, c_out, hw), x.dtype),
        grid=(n // _BB,),
        in_specs=[
            pl.BlockSpec((_BB, c_in, hw), lambda b: (b, 0, 0)),
            pl.BlockSpec((c_in, kh * kw), lambda b: (0, 0)),
            pl.BlockSpec((c_in, 1), lambda b: (0, 0)),
            pl.BlockSpec((c_out, c_in), lambda b: (0, 0)),
            pl.BlockSpec((c_out, 1), lambda b: (0, 0)),
        ],
        out_specs=pl.BlockSpec((_BB, c_out, hw), lambda b: (b, 0, 0)),
        scratch_shapes=[pltpu.VMEM((c_in, hw + 2 * _PAD), jnp.bfloat16)],
        compiler_params=pltpu.CompilerParams(dimension_semantics=("parallel",)),
    )(x_flat, w_taps, b1.reshape(c_in, 1).astype(jnp.bfloat16),
      pw_folded, b2.reshape(c_out, 1))
    return out_flat.reshape(n, c_out, h, w)


# speedup vs baseline: 1.0332x; 1.0332x over previous
import functools

import jax
import jax.numpy as jnp
from jax.experimental import pallas as pl
from jax.experimental.pallas import tpu as pltpu

_BN_EPS = 1e-5
_PAD = 128
_CHUNKS = 2
_BB = 4


def _fused_block_kernel(x_ref, w_ref, b1_ref, pw_ref, b2_ref, o_ref,
                        xpad_ref, *, hw, w_img, kh, kw):
    c = x_ref.shape[1]
    ph, pw_pad = kh // 2, kw // 2
    margin = w_img * ph
    t = hw // _CHUNKS
    wk = t + 2 * margin

    b1 = b1_ref[...]
    wt = w_ref[...]
    pwb = pw_ref[...]
    b2 = b2_ref[...]

    col = jax.lax.broadcasted_iota(jnp.int32, (c, wk), 1) % w_img

    for bb in range(_BB):
        xpad_ref[:, :_PAD] = jnp.zeros((c, _PAD), jnp.bfloat16)
        xpad_ref[:, _PAD + hw:] = jnp.zeros((c, _PAD), jnp.bfloat16)
        xpad_ref[:, _PAD:_PAD + hw] = x_ref[bb].astype(jnp.bfloat16)

        for k in range(_CHUNKS):
            base = _PAD + k * t - margin

            us = []
            for j in range(kw):
                dj = j - pw_pad
                u = xpad_ref[:, base + dj:base + dj + wk]
                if dj < 0:
                    u = jnp.where(col >= -dj, u, jnp.bfloat16(0))
                elif dj > 0:
                    u = jnp.where(col < w_img - dj, u, jnp.bfloat16(0))
                us.append(u)

            acc = None
            for i in range(kh):
                v = None
                for j in range(kw):
                    term = us[j] * wt[:, kw * i + j:kw * i + j + 1]
                    v = term if v is None else v + term
                sh = w_img * i
                part = v[:, sh:sh + t]
                acc = part if acc is None else acc + part

            mid = jnp.maximum(acc + b1, jnp.bfloat16(0))
            y = jnp.dot(pwb, mid, preferred_element_type=jnp.float32)
            o_ref[bb, :, k * t:k * t + t] = (
                jnp.maximum(y + b2, 0.0).astype(o_ref.dtype))


def kernel(x, dw_w, pw_w, bn1_gamma, bn1_beta, bn1_mean, bn1_var,
           bn2_gamma, bn2_beta, bn2_mean, bn2_var):
    n, c_in, h, w = x.shape
    kh, kw = int(dw_w.shape[2]), int(dw_w.shape[3])
    c_out = pw_w.shape[0]
    hw = h * w

    s1 = bn1_gamma / jnp.sqrt(bn1_var + _BN_EPS)
    b1 = bn1_beta - bn1_mean * s1
    s2 = bn2_gamma / jnp.sqrt(bn2_var + _BN_EPS)
    b2 = bn2_beta - bn2_mean * s2
    w_taps = (dw_w.reshape(c_in, kh * kw) * s1[:, None]).astype(jnp.bfloat16)
    pw_folded = (pw_w.reshape(c_out, c_in) * s2[:, None]).astype(jnp.bfloat16)

    x_flat = x.reshape(n, c_in, hw)

    body = functools.partial(_fused_block_kernel, hw=hw, w_img=w, kh=kh, kw=kw)
    out_flat = pl.pallas_call(
        body,
        out_shape=jax.ShapeDtypeStruct((n, c_out, hw), x.dtype),
        grid=(n // _BB,),
        in_specs=[
            pl.BlockSpec((_BB, c_in, hw), lambda b: (b, 0, 0)),
            pl.BlockSpec((c_in, kh * kw), lambda b: (0, 0)),
            pl.BlockSpec((c_in, 1), lambda b: (0, 0)),
            pl.BlockSpec((c_out, c_in), lambda b: (0, 0)),
            pl.BlockSpec((c_out, 1), lambda b: (0, 0)),
        ],
        out_specs=pl.BlockSpec((_BB, c_out, hw), lambda b: (b, 0, 0)),
        scratch_shapes=[pltpu.VMEM((c_in, hw + 2 * _PAD), jnp.bfloat16)],
        compiler_params=pltpu.CompilerParams(dimension_semantics=("parallel",)),
    )(x_flat, w_taps, b1.reshape(c_in, 1).astype(jnp.bfloat16),
      pw_folded, b2.reshape(c_out, 1))
    return out_flat.reshape(n, c_out, h, w)


# speedup vs baseline: 1.0374x; 1.0041x over previous
import functools

import jax
import jax.numpy as jnp
from jax.experimental import pallas as pl
from jax.experimental.pallas import tpu as pltpu

_BN_EPS = 1e-5
_PAD = 128
_CHUNKS = 2
_BB = 4


def _fused_block_kernel(x_ref, w_ref, b1_ref, pw_ref, b2_ref, o_ref,
                        xpad_ref, *, hw, w_img, kh, kw):
    c = x_ref.shape[1]
    ph, pw_pad = kh // 2, kw // 2
    margin = w_img * ph
    t = hw // _CHUNKS
    wk = t + 2 * margin

    b1 = b1_ref[...]
    wt = w_ref[...]
    pwb = pw_ref[...]
    b2 = b2_ref[...]

    col = jax.lax.broadcasted_iota(jnp.int32, (c, wk), 1) % w_img

    xpad_ref[:, :_PAD] = jnp.zeros((c, _PAD), jnp.bfloat16)
    xpad_ref[:, _PAD + hw:] = jnp.zeros((c, _PAD), jnp.bfloat16)

    for bb in range(_BB):
        xpad_ref[:, _PAD:_PAD + hw] = x_ref[bb].astype(jnp.bfloat16)

        for k in range(_CHUNKS):
            base = _PAD + k * t - margin

            us = []
            for j in range(kw):
                dj = j - pw_pad
                u = xpad_ref[:, base + dj:base + dj + wk]
                if dj < 0:
                    u = jnp.where(col >= -dj, u, jnp.bfloat16(0))
                elif dj > 0:
                    u = jnp.where(col < w_img - dj, u, jnp.bfloat16(0))
                us.append(u)

            acc = None
            for i in range(kh):
                v = None
                for j in range(kw):
                    term = us[j] * wt[:, kw * i + j:kw * i + j + 1]
                    v = term if v is None else v + term
                sh = w_img * i
                part = v[:, sh:sh + t]
                acc = part if acc is None else acc + part

            mid = jnp.maximum(acc + b1, jnp.bfloat16(0))
            y = jnp.dot(pwb, mid, preferred_element_type=jnp.float32)
            o_ref[bb, :, k * t:k * t + t] = (
                jnp.maximum(y + b2, 0.0).astype(o_ref.dtype))


def kernel(x, dw_w, pw_w, bn1_gamma, bn1_beta, bn1_mean, bn1_var,
           bn2_gamma, bn2_beta, bn2_mean, bn2_var):
    n, c_in, h, w = x.shape
    kh, kw = int(dw_w.shape[2]), int(dw_w.shape[3])
    c_out = pw_w.shape[0]
    hw = h * w

    s1 = bn1_gamma / jnp.sqrt(bn1_var + _BN_EPS)
    b1 = bn1_beta - bn1_mean * s1
    s2 = bn2_gamma / jnp.sqrt(bn2_var + _BN_EPS)
    b2 = bn2_beta - bn2_mean * s2
    w_taps = (dw_w.reshape(c_in, kh * kw) * s1[:, None]).astype(jnp.bfloat16)
    pw_folded = (pw_w.reshape(c_out, c_in) * s2[:, None]).astype(jnp.bfloat16)

    x_flat = x.reshape(n, c_in, hw)

    body = functools.partial(_fused_block_kernel, hw=hw, w_img=w, kh=kh, kw=kw)
    out_flat = pl.pallas_call(
        body,
        out_shape=jax.ShapeDtypeStruct((n, c_out, hw), x.dtype),
        grid=(n // _BB,),
        in_specs=[
            pl.BlockSpec((_BB, c_in, hw), lambda b: (b, 0, 0)),
            pl.BlockSpec((c_in, kh * kw), lambda b: (0, 0)),
            pl.BlockSpec((c_in, 1), lambda b: (0, 0)),
            pl.BlockSpec((c_out, c_in), lambda b: (0, 0)),
            pl.BlockSpec((c_out, 1), lambda b: (0, 0)),
        ],
        out_specs=pl.BlockSpec((_BB, c_out, hw), lambda b: (b, 0, 0)),
        scratch_shapes=[pltpu.VMEM((c_in, hw + 2 * _PAD), jnp.bfloat16)],
        compiler_params=pltpu.CompilerParams(dimension_semantics=("parallel",)),
    )(x_flat, w_taps, b1.reshape(c_in, 1).astype(jnp.bfloat16),
      pw_folded, b2.reshape(c_out, 1))
    return out_flat.reshape(n, c_out, h, w)
